# baseline (speedup 1.0000x reference)
import functools

import jax
import jax.numpy as jnp
from jax import lax
from jax.experimental import pallas as pl
from jax.experimental.pallas import tpu as pltpu
from jax.experimental.pallas import tpu_sc as plsc

B = 128
V = 100000
C = 0.1
W = 2048
K = (V + W - 1) // W
NWORK = B // 16


def _sc_gather_body(target_hbm, pred_flat_hbm, out_hbm, tgt_v, idx_v, val_v, sem):
    wid = lax.axis_index("s") * 2 + lax.axis_index("c")

    @pl.when(wid < NWORK)
    def _():
        base = wid * 16
        pltpu.sync_copy(target_hbm.at[pl.ds(base, 16)], tgt_v)
        rows = base + lax.iota(jnp.int32, 16)
        idx_v[...] = tgt_v[...] + rows * V
        pltpu.async_copy(pred_flat_hbm.at[idx_v], val_v, sem).wait()
        pltpu.sync_copy(val_v, out_hbm.at[pl.ds(base, 16)])


_sc_gather = functools.partial(
    pl.kernel,
    mesh=plsc.VectorSubcoreMesh(core_axis_name="c", subcore_axis_name="s"),
    out_type=jax.ShapeDtypeStruct((B,), jnp.float32),
    scratch_types=[
        pltpu.VMEM((16,), jnp.int32),
        pltpu.VMEM((16,), jnp.int32),
        pltpu.VMEM((16,), jnp.float32),
        pltpu.SemaphoreType.DMA,
    ],
)(_sc_gather_body)


def _tc_body(correct_ref, pred_ref, out_ref, acc_ref):
    k = pl.program_id(0)

    @pl.when(k == 0)
    def _():
        acc_ref[...] = jnp.zeros_like(acc_ref)

    x = pred_ref[...]
    corr = correct_ref[...]
    t = jnp.maximum(x - corr + C, 0.0)
    cols = k * W + lax.broadcasted_iota(jnp.int32, (B, W), 1)
    acc_ref[...] += jnp.where(cols < V, t, 0.0)

    @pl.when(k == K - 1)
    def _():
        out_ref[0] = (jnp.sum(acc_ref[...]) - B * C) / B


def kernel(target, prediction):
    target = target.astype(jnp.int32)
    correct = _sc_gather(target, prediction.reshape(-1))
    out = pl.pallas_call(
        _tc_body,
        grid=(K,),
        in_specs=[
            pl.BlockSpec((B, 1), lambda k: (0, 0)),
            pl.BlockSpec((B, W), lambda k: (0, k)),
        ],
        out_specs=pl.BlockSpec(memory_space=pltpu.SMEM),
        out_shape=jax.ShapeDtypeStruct((1,), jnp.float32),
        scratch_shapes=[pltpu.VMEM((B, W), jnp.float32)],
    )(correct.reshape(B, 1), prediction)
    return out


# speedup vs baseline: 2.0008x; 2.0008x over previous
import functools

import jax
import jax.numpy as jnp
from jax import lax
from jax.experimental import pallas as pl
from jax.experimental.pallas import tpu as pltpu
from jax.experimental.pallas import tpu_sc as plsc

B = 128
V = 100000
C = 0.1
W = 8192
K = (V + W - 1) // W
S = W // 4
NWORK = B // 16


def _sc_gather_body(target_hbm, pred_hbm, out_hbm, tgt_v, vals_v, diag_v, sem):
    wid = lax.axis_index("s") * 2 + lax.axis_index("c")

    @pl.when(wid < NWORK)
    def _():
        base = wid * 16
        pltpu.sync_copy(target_hbm.at[pl.ds(base, 16)], tgt_v)
        tv = tgt_v[...]
        handles = []
        offs = []
        for i in range(16):
            t = tv[i]
            cs = (t // 16) * 16
            handles.append(
                pltpu.async_copy(
                    pred_hbm.at[base + i, pl.ds(cs, 16)], vals_v.at[i], sem
                )
            )
            offs.append(t - cs)
        for h in handles:
            h.wait()
        ii = lax.iota(jnp.int32, 16)
        d = jnp.full((16,), -C, jnp.float32)
        for i in range(16):
            off = jnp.full((16,), offs[i], jnp.int32)
            g = vals_v[i, :].at[off].get(mode="promise_in_bounds")
            d = jnp.where(ii == i, g - C, d)
        diag_v[...] = d
        pltpu.sync_copy(diag_v, out_hbm.at[pl.ds(base, 16)])


_sc_gather = functools.partial(
    pl.kernel,
    mesh=plsc.VectorSubcoreMesh(core_axis_name="c", subcore_axis_name="s"),
    out_type=jax.ShapeDtypeStruct((B,), jnp.float32),
    scratch_types=[
        pltpu.VMEM((16,), jnp.int32),
        pltpu.VMEM((16, 16), jnp.float32),
        pltpu.VMEM((16,), jnp.float32),
        pltpu.SemaphoreType.DMA,
    ],
)(_sc_gather_body)


def _tc_body(corrc_ref, pred_ref, out_ref, acc_ref):
    k = pl.program_id(0)

    @pl.when(k == 0)
    def _():
        acc_ref[...] = jnp.zeros_like(acc_ref)

    x = pred_ref[...]
    t = jnp.maximum(x - corrc_ref[...], 0.0)

    @pl.when(k < K - 1)
    def _():
        acc_ref[...] += (t[:, :S] + t[:, S:2 * S]) + (t[:, 2 * S:3 * S] + t[:, 3 * S:])

    @pl.when(k == K - 1)
    def _():
        cols = k * W + lax.broadcasted_iota(jnp.int32, (B, W), 1)
        tm = jnp.where(cols < V, t, 0.0)
        acc_ref[...] += (tm[:, :S] + tm[:, S:2 * S]) + (tm[:, 2 * S:3 * S] + tm[:, 3 * S:])
        out_ref[0] = (jnp.sum(acc_ref[...]) - B * C) / B


def kernel(target, prediction):
    target = target.astype(jnp.int32)
    corrc = _sc_gather(target, prediction)
    out = pl.pallas_call(
        _tc_body,
        grid=(K,),
        in_specs=[
            pl.BlockSpec((B, 1), lambda k: (0, 0)),
            pl.BlockSpec((B, W), lambda k: (0, k)),
        ],
        out_specs=pl.BlockSpec(memory_space=pltpu.SMEM),
        out_shape=jax.ShapeDtypeStruct((1,), jnp.float32),
        scratch_shapes=[pltpu.VMEM((B, S), jnp.float32)],
    )(corrc.reshape(B, 1), prediction)
    return out
